# baseline (speedup 1.0000x reference)
import functools

import jax
import jax.numpy as jnp
from jax import lax
from jax.experimental import pallas as pl
from jax.experimental.pallas import tpu as pltpu
from jax.experimental.pallas import tpu_sc as plsc

BATCH = 4096
HIDDEN = 64
LANES = 128


def _make_sc_gather(V, D, B):
    info = plsc.get_sparse_core_info()
    NC, NS = info.num_cores, info.num_subcores
    NW = NC * NS
    assert B % (8 * NW) == 0
    b_per_w = B // NW
    L = 16
    mesh = plsc.VectorSubcoreMesh(core_axis_name="c", subcore_axis_name="s")

    @functools.partial(
        pl.kernel,
        mesh=mesh,
        out_type=jax.ShapeDtypeStruct((B, D), jnp.float32),
        scratch_types=[
            pltpu.VMEM((b_per_w,), jnp.int32),
            pltpu.VMEM((D, LANES), jnp.float32),
            pltpu.VMEM((D, LANES), jnp.float32),
            pltpu.VMEM((D, LANES), jnp.float32),
            pltpu.VMEM((D, LANES), jnp.float32),
            pltpu.VMEM((D, LANES), jnp.float32),
            pltpu.VMEM((D, LANES), jnp.float32),
            pltpu.VMEM((D, LANES), jnp.float32),
            pltpu.VMEM((D, LANES), jnp.float32),
            pltpu.VMEM((D, LANES), jnp.float32),
            pltpu.VMEM((D, LANES), jnp.float32),
            pltpu.VMEM((D, LANES), jnp.float32),
            pltpu.VMEM((D, LANES), jnp.float32),
            pltpu.VMEM((b_per_w, D), jnp.float32),
            pltpu.SemaphoreType.DMA,
            pltpu.SemaphoreType.DMA,
            pltpu.SemaphoreType.DMA,
            pltpu.SemaphoreType.DMA,
            pltpu.SemaphoreType.DMA,
            pltpu.SemaphoreType.DMA,
            pltpu.SemaphoreType.DMA,
            pltpu.SemaphoreType.DMA,
            pltpu.SemaphoreType.DMA,
            pltpu.SemaphoreType.DMA,
            pltpu.SemaphoreType.DMA,
            pltpu.SemaphoreType.DMA,
        ],
        compiler_params=pltpu.CompilerParams(needs_layout_passes=False),
    )
    def gather(tablet_hbm, idx_hbm, out_hbm, idx_v, buf0, buf1, buf2, buf3,
               buf4, buf5, buf6, buf7, buf8, buf9, buf10, buf11, rows_v,
               sem0, sem1, sem2, sem3, sem4, sem5, sem6, sem7, sem8, sem9,
               sem10, sem11):
        NBUF = 12
        wid = lax.axis_index("s") * NC + lax.axis_index("c")
        base = wid * b_per_w
        pltpu.sync_copy(idx_hbm.at[pl.ds(base, b_per_w)], idx_v)
        lane = lax.iota(jnp.int32, L)
        bufs = (buf0, buf1, buf2, buf3, buf4, buf5, buf6, buf7, buf8, buf9, buf10, buf11)
        sems = (sem0, sem1, sem2, sem3, sem4, sem5, sem6, sem7, sem8, sem9, sem10, sem11)

        def fetch(j):
            vec = idx_v[pl.ds((j // L) * L, L)]
            i = jnp.sum(jnp.where(lane == (j % L), vec, 0))
            off = pl.multiple_of((i >> 7) * LANES, LANES)
            d = pltpu.make_async_copy(
                tablet_hbm.at[:, pl.ds(off, LANES)], bufs[j % NBUF],
                sems[j % NBUF])
            d.start()
            return i, d

        pending = [fetch(j) for j in range(NBUF - 1)]
        for j in range(b_per_w):
            i, d = pending.pop(0)
            if j + NBUF - 1 < b_per_w:
                pending.append(fetch(j + NBUF - 1))
            d.wait()
            r = jnp.full((L,), i & (LANES - 1), dtype=jnp.int32)
            buf = bufs[j % NBUF]
            for q in range(D // L):
                vals = plsc.load_gather(
                    buf, [lax.iota(jnp.int32, L) + q * L, r])
                rows_v[j, pl.ds(q * L, L)] = vals
        pltpu.sync_copy(rows_v, out_hbm.at[pl.ds(base, b_per_w)])

    return gather


def _gru_body(x_ref, h_ref, wih_ref, whh_ref, bih_ref, bhh_ref, out_ref,
              hid_ref):
    H = HIDDEN
    x = x_ref[...]
    h = h_ref[0]
    dims = (((1,), (1,)), ((), ()))
    gi = lax.dot_general(x, wih_ref[...], dims,
                         preferred_element_type=jnp.float32)
    gh = lax.dot_general(h, whh_ref[...], dims,
                         preferred_element_type=jnp.float32)
    gi = gi + bih_ref[...].reshape(1, 3 * H)
    gh = gh + bhh_ref[...].reshape(1, 3 * H)
    r = jax.nn.sigmoid(gi[:, :H] + gh[:, :H])
    z = jax.nn.sigmoid(gi[:, H:2 * H] + gh[:, H:2 * H])
    n = jnp.tanh(gi[:, 2 * H:] + r * gh[:, 2 * H:])
    h1 = (1.0 - z) * n + z * h
    out_ref[0] = h1
    hid_ref[0] = h1


def kernel(input_data, batch_size, hidden, embedding_matrix, W_ih, W_hh,
           b_ih, b_hh):
    V, D = embedding_matrix.shape
    idx = input_data.astype(jnp.int32)
    tablet = embedding_matrix.T

    gather = _make_sc_gather(V, D, BATCH)
    x = gather(tablet, idx)

    out, hid = pl.pallas_call(
        _gru_body,
        out_shape=(
            jax.ShapeDtypeStruct((1, BATCH, HIDDEN), jnp.float32),
            jax.ShapeDtypeStruct((1, BATCH, HIDDEN), jnp.float32),
        ),
    )(x, hidden, W_ih, W_hh, b_ih, b_hh)
    return (out, hid)
